# baseline (speedup 1.0000x reference)
import functools

import jax
import jax.numpy as jnp
from jax import lax
from jax.experimental import pallas as pl
from jax.experimental.pallas import tpu as pltpu
from jax.experimental.pallas import tpu_sc as plsc

N = 10000
E = 320000
DH = 128
DOUT = 64
EPS = 1e-5

NCORE = 2
NSUB = 16
NW = NCORE * NSUB
CHUNK = 128
NBUF = 2
EPT = 10240
EPAD = EPT * NW
NCHUNKS = EPT // CHUNK
RPT = 624
TAIL = N - RPT * NSUB
ACCR = N + 16
ZC = 16
DR = 104
DSP = 64


@functools.lru_cache(maxsize=None)
def _make_seg_sum(D, nsplit):
  mesh = plsc.VectorSubcoreMesh(core_axis_name="c", subcore_axis_name="s")

  @functools.partial(
      pl.kernel,
      out_type=jax.ShapeDtypeStruct((NCORE, nsplit, N, D), jnp.float32),
      mesh=mesh,
      scratch_types=[
          pltpu.VMEM((NCHUNKS, CHUNK), jnp.int32),
          pltpu.VMEM((NCHUNKS, CHUNK), jnp.int32),
          [pltpu.VMEM((CHUNK, D), jnp.float32)] * NBUF,
          pltpu.VMEM((ZC, D), jnp.float32),
          pltpu.VMEM((DR, D), jnp.float32),
          pltpu.VMEM_SHARED((N, D), jnp.float32),
          pltpu.VMEM_SHARED((ACCR, D), jnp.float32),
          [pltpu.SemaphoreType.DMA] * NBUF,
          [pltpu.SemaphoreType.DMA] * NBUF,
      ],
      compiler_params=pltpu.CompilerParams(use_tc_tiling_on_sc=False),
  )
  def kern(table3, src, dst, out, sidx, didx, rows, zbuf, stage, tbl, acc,
           sems, ssems):
    c = lax.axis_index("c")
    s = lax.axis_index("s")
    wid = c * NSUB + s
    last = s == NSUB - 1
    r0 = s * RPT

    pltpu.sync_copy(src.at[wid], sidx)
    pltpu.sync_copy(dst.at[wid], didx)

    def _zrow(i, carry):
      for j in range(D // 16):
        zbuf[i, pl.ds(j * 16, 16)] = jnp.zeros((16,), jnp.float32)
      return carry
    lax.fori_loop(0, ZC, _zrow, 0)

    def _half(h, hcarry):
      table = table3.at[h]

      def _st(k, carry):
        rr = r0 + k * DR
        pltpu.sync_copy(table.at[pl.ds(rr, DR)], stage)
        pltpu.sync_copy(stage, tbl.at[pl.ds(rr, DR)])
        return carry
      lax.fori_loop(0, RPT // DR, _st, 0)

      @pl.when(last)
      def _():
        pltpu.sync_copy(table.at[pl.ds(RPT * NSUB, TAIL)],
                        stage.at[pl.ds(0, TAIL)])
        pltpu.sync_copy(stage.at[pl.ds(0, TAIL)],
                        tbl.at[pl.ds(RPT * NSUB, TAIL)])

      def _zcp(k, carry):
        pltpu.sync_copy(zbuf, acc.at[pl.ds(r0 + k * ZC, ZC)])
        return carry
      lax.fori_loop(0, RPT // ZC, _zcp, 0)

      @pl.when(last)
      def _():
        pltpu.sync_copy(zbuf, acc.at[pl.ds(RPT * NSUB, TAIL)])
      plsc.subcore_barrier()

      for b in range(NBUF):
        pltpu.async_copy(tbl.at[sidx.at[b]], rows[b], sems[b])

      def _pipe(it, carry):
        for b in range(NBUF):
          ch = it * NBUF + b
          pltpu.make_async_copy(tbl.at[sidx.at[0]], rows[b], sems[b]).wait()
          pltpu.async_copy(rows[b], acc.at[didx.at[ch]], ssems[b], add=True)
          bp = (b - 1) % NBUF

          @pl.when((ch >= 1) & (ch - 1 + NBUF < NCHUNKS))
          def _():
            pltpu.make_async_copy(
                rows[bp], acc.at[didx.at[0]], ssems[bp]).wait()
            pltpu.async_copy(
                tbl.at[sidx.at[ch - 1 + NBUF]], rows[bp], sems[bp])
        return carry
      lax.fori_loop(0, NCHUNKS // NBUF, _pipe, 0)
      for b in range(NBUF):
        pltpu.make_async_copy(rows[b], acc.at[didx.at[0]], ssems[b]).wait()
      plsc.subcore_barrier()

      def _dr(k, carry):
        rr = r0 + k * DR
        pltpu.sync_copy(acc.at[pl.ds(rr, DR)], stage)
        pltpu.sync_copy(stage, out.at[c, h, pl.ds(rr, DR)])
        return carry
      lax.fori_loop(0, RPT // DR, _dr, 0)

      @pl.when(last)
      def _():
        pltpu.sync_copy(acc.at[pl.ds(RPT * NSUB, TAIL)],
                        stage.at[pl.ds(0, TAIL)])
        pltpu.sync_copy(stage.at[pl.ds(0, TAIL)],
                        out.at[c, h, pl.ds(RPT * NSUB, TAIL)])
      return hcarry
    lax.fori_loop(0, nsplit, _half, 0)

  return kern


@functools.lru_cache(maxsize=None)
def _make_deg():
  D = 16
  mesh = plsc.VectorSubcoreMesh(core_axis_name="c", subcore_axis_name="s")

  @functools.partial(
      pl.kernel,
      out_type=jax.ShapeDtypeStruct((NCORE, N, D), jnp.float32),
      mesh=mesh,
      scratch_types=[
          pltpu.VMEM((NCHUNKS, CHUNK), jnp.int32),
          pltpu.VMEM((CHUNK, D), jnp.float32),
          pltpu.VMEM((ZC, D), jnp.float32),
          pltpu.VMEM_SHARED((ACCR, D), jnp.float32),
          pltpu.SemaphoreType.DMA,
      ],
      compiler_params=pltpu.CompilerParams(use_tc_tiling_on_sc=False),
  )
  def kern(dst, out, didx, ones_r, zbuf, acc, sem):
    c = lax.axis_index("c")
    s = lax.axis_index("s")
    wid = c * NSUB + s
    last = s == NSUB - 1
    r0 = s * RPT

    pltpu.sync_copy(dst.at[wid], didx)

    def _fill(i, carry):
      zbuf[i, pl.ds(0, 16)] = jnp.zeros((16,), jnp.float32)
      return carry
    lax.fori_loop(0, ZC, _fill, 0)

    def _fill1(i, carry):
      ones_r[i, pl.ds(0, 16)] = jnp.ones((16,), jnp.float32)
      return carry
    lax.fori_loop(0, CHUNK, _fill1, 0)

    def _zcp(k, carry):
      pltpu.sync_copy(zbuf, acc.at[pl.ds(r0 + k * ZC, ZC)])
      return carry
    lax.fori_loop(0, RPT // ZC, _zcp, 0)

    @pl.when(last)
    def _():
      pltpu.sync_copy(zbuf, acc.at[pl.ds(RPT * NSUB, TAIL)])
    plsc.subcore_barrier()

    FIRE = 16

    def _batch(g, carry):
      for b in range(FIRE):
        pltpu.async_copy(ones_r, acc.at[didx.at[g * FIRE + b]], sem,
                         add=True)
      for b in range(FIRE):
        pltpu.make_async_copy(ones_r, acc.at[didx.at[0]], sem).wait()
      return carry
    lax.fori_loop(0, NCHUNKS // FIRE, _batch, 0)
    plsc.subcore_barrier()

    def _dr(k, carry):
      rr = r0 + k * ZC
      pltpu.sync_copy(acc.at[pl.ds(rr, ZC)], zbuf)
      pltpu.sync_copy(zbuf, out.at[c, pl.ds(rr, ZC)])
      return carry
    lax.fori_loop(0, RPT // ZC, _dr, 0)

    @pl.when(last)
    def _():
      pltpu.sync_copy(acc.at[pl.ds(RPT * NSUB, TAIL)], zbuf)
      pltpu.sync_copy(zbuf, out.at[c, pl.ds(RPT * NSUB, TAIL)])

  return kern


BN = 1000
NB = N // BN


def _tc_pre(degp, x, W0):
  def body(deg_ref, x_ref, w_ref, g_ref, dinv_ref):
    deg = deg_ref[0, :, 0:1] + deg_ref[1, :, 0:1] + 1.0
    dinv = lax.rsqrt(deg)
    h = jnp.dot(x_ref[...], w_ref[...], preferred_element_type=jnp.float32)
    gs = dinv * h
    g_ref[0] = gs[:, :DSP]
    g_ref[1] = gs[:, DSP:]
    dinv_ref[...] = dinv

  return pl.pallas_call(
      body,
      grid=(NB,),
      in_specs=[
          pl.BlockSpec((NCORE, BN, 16), lambda j: (0, j, 0)),
          pl.BlockSpec((BN, DH), lambda j: (j, 0)),
          pl.BlockSpec((DH, DH), lambda j: (0, 0)),
      ],
      out_specs=[
          pl.BlockSpec((2, BN, DSP), lambda j: (0, j, 0)),
          pl.BlockSpec((BN, 1), lambda j: (j, 0)),
      ],
      out_shape=[
          jax.ShapeDtypeStruct((2, N, DSP), jnp.float32),
          jax.ShapeDtypeStruct((N, 1), jnp.float32),
      ],
  )(degp, x, W0)


def _make_tc_norm(DO, final):
  def body(p_ref, g_ref, dinv_ref, b_ref, gw_ref, gb_ref, ga_ref, w_ref,
           wb_ref, out_ref, S1, S2):
    ph = pl.program_id(0)
    j = pl.program_id(1)
    dinv = dinv_ref[...]
    psum = p_ref[0] + p_ref[1]
    pcat = jnp.concatenate([psum[0] + g_ref[0], psum[1] + g_ref[1]], axis=1)
    t = dinv * pcat + b_ref[...]

    @pl.when(ph == 0)
    def _():
      s1 = jnp.sum(t, axis=0, keepdims=True)
      s2 = jnp.sum(t * t, axis=0, keepdims=True)

      @pl.when(j == 0)
      def _():
        S1[...] = s1
        S2[...] = s2

      @pl.when(j != 0)
      def _():
        S1[...] += s1
        S2[...] += s2

    @pl.when(ph == 1)
    def _():
      m = S1[...] * (1.0 / N)
      ex2 = S2[...] * (1.0 / N)
      am = ga_ref[...] * m
      var = ex2 - 2.0 * m * am + am * am
      y = gw_ref[...] * (t - am) * lax.rsqrt(var + EPS) + gb_ref[...]
      y = jnp.maximum(y, 0.0)
      h = jnp.dot(y, w_ref[...], preferred_element_type=jnp.float32)
      if final:
        out_ref[...] = h + wb_ref[...]
      else:
        gs = dinv * h
        out_ref[0] = gs[:, :DSP]
        out_ref[1] = gs[:, DSP:]

  def run(p, g, dinv, b, gw, gb, ga, W, wb):
    if final:
      out_spec = pl.BlockSpec((BN, DO), lambda ph, j: (j, 0))
      out_shape = jax.ShapeDtypeStruct((N, DO), jnp.float32)
    else:
      out_spec = pl.BlockSpec((2, BN, DSP), lambda ph, j: (0, j, 0))
      out_shape = jax.ShapeDtypeStruct((2, N, DSP), jnp.float32)
    return pl.pallas_call(
        body,
        grid=(2, NB),
        in_specs=[
            pl.BlockSpec((NCORE, 2, BN, DSP), lambda ph, j: (0, 0, j, 0)),
            pl.BlockSpec((2, BN, DSP), lambda ph, j: (0, j, 0)),
            pl.BlockSpec((BN, 1), lambda ph, j: (j, 0)),
            pl.BlockSpec((1, DH), lambda ph, j: (0, 0)),
            pl.BlockSpec((1, DH), lambda ph, j: (0, 0)),
            pl.BlockSpec((1, DH), lambda ph, j: (0, 0)),
            pl.BlockSpec((1, DH), lambda ph, j: (0, 0)),
            pl.BlockSpec((DH, DO), lambda ph, j: (0, 0)),
            pl.BlockSpec((1, DO), lambda ph, j: (0, 0)),
        ],
        out_specs=out_spec,
        out_shape=out_shape,
        scratch_shapes=[
            pltpu.VMEM((1, DH), jnp.float32),
            pltpu.VMEM((1, DH), jnp.float32),
        ],
    )(p, g, dinv, b.reshape(1, DH), gw.reshape(1, DH), gb.reshape(1, DH),
      ga.reshape(1, DH), W, wb.reshape(1, DO))

  return run


_tc_mid = _make_tc_norm(DH, final=False)
_tc_fin = _make_tc_norm(DOUT, final=True)


def kernel(x, edge_index, W0, b0, gn_w0, gn_b0, gn_a0, W1, b1, gn_w1, gn_b1,
           gn_a1, W2, b2, gn_w2, gn_b2, gn_a2, fc_W, fc_b):
  src = edge_index[0]
  dst = edge_index[1]
  pad = EPAD - E
  src_p = jnp.concatenate([src, jnp.zeros((pad,), jnp.int32)]).reshape(
      NW, NCHUNKS, CHUNK)
  dst_p = jnp.concatenate([dst, jnp.full((pad,), N, jnp.int32)]).reshape(
      NW, NCHUNKS, CHUNK)

  degp = _make_deg()(dst_p)

  g0, dinv = _tc_pre(degp, x, W0)

  spmm = _make_seg_sum(DSP, DH // DSP)
  bS = jnp.stack([b0, b1])
  gwS = jnp.stack([gn_w0, gn_w1])
  gbS = jnp.stack([gn_b0, gn_b1])
  gaS = jnp.stack([gn_a0, gn_a1])
  WS = jnp.stack([W1, W2])

  def body(i, carry):
    g, _ = carry
    p = spmm(g, src_p, dst_p)
    k = jnp.minimum(i, 1)
    g_new = lax.cond(
        i < 2,
        lambda: _tc_mid(p, g, dinv, bS[k], gwS[k], gbS[k], gaS[k], WS[k],
                        bS[k]),
        lambda: g,
    )
    return (g_new, p)

  g2, p = lax.fori_loop(
      0, 3, body, (g0, jnp.zeros((NCORE, 2, N, DSP), jnp.float32)))
  return _tc_fin(p, g2, dinv, b2, gn_w2, gn_b2, gn_a2, fc_W, fc_b)


# speedup vs baseline: 1.0186x; 1.0186x over previous
import functools

import jax
import jax.numpy as jnp
from jax import lax
from jax.experimental import pallas as pl
from jax.experimental.pallas import tpu as pltpu
from jax.experimental.pallas import tpu_sc as plsc

N = 10000
E = 320000
DH = 128
DOUT = 64
EPS = 1e-5

NCORE = 2
NSUB = 16
NW = NCORE * NSUB
CHUNK = 128
NBUF = 2
EPT = 10240
EPAD = EPT * NW
NCHUNKS = EPT // CHUNK
RPT = 624
TAIL = N - RPT * NSUB
ACCR = N + 16
ZC = 16
DR = 104
DSP = 64


@functools.lru_cache(maxsize=None)
def _make_seg_sum(D, nsplit):
  mesh = plsc.VectorSubcoreMesh(core_axis_name="c", subcore_axis_name="s")

  @functools.partial(
      pl.kernel,
      out_type=jax.ShapeDtypeStruct((NCORE, nsplit, N, D), jnp.float32),
      mesh=mesh,
      scratch_types=[
          pltpu.VMEM((NCHUNKS, CHUNK), jnp.int32),
          pltpu.VMEM((NCHUNKS, CHUNK), jnp.int32),
          [pltpu.VMEM((CHUNK, D), jnp.float32)] * NBUF,
          pltpu.VMEM((ZC, D), jnp.float32),
          pltpu.VMEM((DR, D), jnp.float32),
          pltpu.VMEM_SHARED((N, D), jnp.float32),
          pltpu.VMEM_SHARED((ACCR, D), jnp.float32),
          [pltpu.SemaphoreType.DMA] * NBUF,
          [pltpu.SemaphoreType.DMA] * NBUF,
      ],
      compiler_params=pltpu.CompilerParams(use_tc_tiling_on_sc=False),
  )
  def kern(table3, src, dst, out, sidx, didx, rows, zbuf, stage, tbl, acc,
           sems, ssems):
    c = lax.axis_index("c")
    s = lax.axis_index("s")
    wid = c * NSUB + s
    last = s == NSUB - 1
    r0 = s * RPT

    pltpu.sync_copy(src.at[wid], sidx)
    pltpu.sync_copy(dst.at[wid], didx)

    def _zrow(i, carry):
      for j in range(D // 16):
        zbuf[i, pl.ds(j * 16, 16)] = jnp.zeros((16,), jnp.float32)
      return carry
    lax.fori_loop(0, ZC, _zrow, 0)

    for h in range(nsplit):
      table = table3.at[h]

      def _st(k, carry):
        rr = r0 + k * DR
        pltpu.sync_copy(table.at[pl.ds(rr, DR)], stage)
        pltpu.sync_copy(stage, tbl.at[pl.ds(rr, DR)])
        return carry
      lax.fori_loop(0, RPT // DR, _st, 0)

      @pl.when(last)
      def _():
        pltpu.sync_copy(table.at[pl.ds(RPT * NSUB, TAIL)],
                        stage.at[pl.ds(0, TAIL)])
        pltpu.sync_copy(stage.at[pl.ds(0, TAIL)],
                        tbl.at[pl.ds(RPT * NSUB, TAIL)])

      def _zcp(k, carry):
        pltpu.sync_copy(zbuf, acc.at[pl.ds(r0 + k * ZC, ZC)])
        return carry
      lax.fori_loop(0, RPT // ZC, _zcp, 0)

      @pl.when(last)
      def _():
        pltpu.sync_copy(zbuf, acc.at[pl.ds(RPT * NSUB, TAIL)])
      plsc.subcore_barrier()

      for b in range(NBUF):
        pltpu.async_copy(tbl.at[sidx.at[b]], rows[b], sems[b])

      def _pipe(it, carry):
        for b in range(NBUF):
          ch = it * NBUF + b
          pltpu.make_async_copy(tbl.at[sidx.at[0]], rows[b], sems[b]).wait()
          pltpu.async_copy(rows[b], acc.at[didx.at[ch]], ssems[b], add=True)
          bp = (b - 1) % NBUF

          @pl.when((ch >= 1) & (ch - 1 + NBUF < NCHUNKS))
          def _():
            pltpu.make_async_copy(
                rows[bp], acc.at[didx.at[0]], ssems[bp]).wait()
            pltpu.async_copy(
                tbl.at[sidx.at[ch - 1 + NBUF]], rows[bp], sems[bp])
        return carry
      lax.fori_loop(0, NCHUNKS // NBUF, _pipe, 0)
      for b in range(NBUF):
        pltpu.make_async_copy(rows[b], acc.at[didx.at[0]], ssems[b]).wait()
      plsc.subcore_barrier()

      def _dr(k, carry):
        rr = r0 + k * DR
        pltpu.sync_copy(acc.at[pl.ds(rr, DR)], stage)
        pltpu.sync_copy(stage, out.at[c, h, pl.ds(rr, DR)])
        return carry
      lax.fori_loop(0, RPT // DR, _dr, 0)

      @pl.when(last)
      def _():
        pltpu.sync_copy(acc.at[pl.ds(RPT * NSUB, TAIL)],
                        stage.at[pl.ds(0, TAIL)])
        pltpu.sync_copy(stage.at[pl.ds(0, TAIL)],
                        out.at[c, h, pl.ds(RPT * NSUB, TAIL)])

  return kern


@functools.lru_cache(maxsize=None)
def _make_deg():
  D = 16
  mesh = plsc.VectorSubcoreMesh(core_axis_name="c", subcore_axis_name="s")

  @functools.partial(
      pl.kernel,
      out_type=jax.ShapeDtypeStruct((NCORE, N, D), jnp.float32),
      mesh=mesh,
      scratch_types=[
          pltpu.VMEM((NCHUNKS, CHUNK), jnp.int32),
          pltpu.VMEM((CHUNK, D), jnp.float32),
          pltpu.VMEM((ZC, D), jnp.float32),
          pltpu.VMEM_SHARED((ACCR, D), jnp.float32),
          pltpu.SemaphoreType.DMA,
      ],
      compiler_params=pltpu.CompilerParams(use_tc_tiling_on_sc=False),
  )
  def kern(dst, out, didx, ones_r, zbuf, acc, sem):
    c = lax.axis_index("c")
    s = lax.axis_index("s")
    wid = c * NSUB + s
    last = s == NSUB - 1
    r0 = s * RPT

    pltpu.sync_copy(dst.at[wid], didx)

    def _fill(i, carry):
      zbuf[i, pl.ds(0, 16)] = jnp.zeros((16,), jnp.float32)
      return carry
    lax.fori_loop(0, ZC, _fill, 0)

    def _fill1(i, carry):
      ones_r[i, pl.ds(0, 16)] = jnp.ones((16,), jnp.float32)
      return carry
    lax.fori_loop(0, CHUNK, _fill1, 0)

    def _zcp(k, carry):
      pltpu.sync_copy(zbuf, acc.at[pl.ds(r0 + k * ZC, ZC)])
      return carry
    lax.fori_loop(0, RPT // ZC, _zcp, 0)

    @pl.when(last)
    def _():
      pltpu.sync_copy(zbuf, acc.at[pl.ds(RPT * NSUB, TAIL)])
    plsc.subcore_barrier()

    FIRE = 16

    def _batch(g, carry):
      for b in range(FIRE):
        pltpu.async_copy(ones_r, acc.at[didx.at[g * FIRE + b]], sem,
                         add=True)
      for b in range(FIRE):
        pltpu.make_async_copy(ones_r, acc.at[didx.at[0]], sem).wait()
      return carry
    lax.fori_loop(0, NCHUNKS // FIRE, _batch, 0)
    plsc.subcore_barrier()

    def _dr(k, carry):
      rr = r0 + k * ZC
      pltpu.sync_copy(acc.at[pl.ds(rr, ZC)], zbuf)
      pltpu.sync_copy(zbuf, out.at[c, pl.ds(rr, ZC)])
      return carry
    lax.fori_loop(0, RPT // ZC, _dr, 0)

    @pl.when(last)
    def _():
      pltpu.sync_copy(acc.at[pl.ds(RPT * NSUB, TAIL)], zbuf)
      pltpu.sync_copy(zbuf, out.at[c, pl.ds(RPT * NSUB, TAIL)])

  return kern


BN = 2000
NB = N // BN


def _tc_pre(degp, x, W0):
  def body(deg_ref, x_ref, w_ref, g_ref, dinv_ref):
    deg = deg_ref[0, :, 0:1] + deg_ref[1, :, 0:1] + 1.0
    dinv = lax.rsqrt(deg)
    h = jnp.dot(x_ref[...], w_ref[...], preferred_element_type=jnp.float32)
    gs = dinv * h
    g_ref[0] = gs[:, :DSP]
    g_ref[1] = gs[:, DSP:]
    dinv_ref[...] = dinv

  return pl.pallas_call(
      body,
      grid=(NB,),
      in_specs=[
          pl.BlockSpec((NCORE, BN, 16), lambda j: (0, j, 0)),
          pl.BlockSpec((BN, DH), lambda j: (j, 0)),
          pl.BlockSpec((DH, DH), lambda j: (0, 0)),
      ],
      out_specs=[
          pl.BlockSpec((2, BN, DSP), lambda j: (0, j, 0)),
          pl.BlockSpec((BN, 1), lambda j: (j, 0)),
      ],
      out_shape=[
          jax.ShapeDtypeStruct((2, N, DSP), jnp.float32),
          jax.ShapeDtypeStruct((N, 1), jnp.float32),
      ],
  )(degp, x, W0)


def _make_tc_norm(DO, final):
  def body(p_ref, g_ref, dinv_ref, b_ref, gw_ref, gb_ref, ga_ref, w_ref,
           wb_ref, out_ref, S1, S2):
    ph = pl.program_id(0)
    j = pl.program_id(1)
    dinv = dinv_ref[...]
    psum = p_ref[0] + p_ref[1]
    pcat = jnp.concatenate([psum[0] + g_ref[0], psum[1] + g_ref[1]], axis=1)
    t = dinv * pcat + b_ref[...]

    @pl.when(ph == 0)
    def _():
      s1 = jnp.sum(t, axis=0, keepdims=True)
      s2 = jnp.sum(t * t, axis=0, keepdims=True)

      @pl.when(j == 0)
      def _():
        S1[...] = s1
        S2[...] = s2

      @pl.when(j != 0)
      def _():
        S1[...] += s1
        S2[...] += s2

    @pl.when(ph == 1)
    def _():
      m = S1[...] * (1.0 / N)
      ex2 = S2[...] * (1.0 / N)
      am = ga_ref[...] * m
      var = ex2 - 2.0 * m * am + am * am
      y = gw_ref[...] * (t - am) * lax.rsqrt(var + EPS) + gb_ref[...]
      y = jnp.maximum(y, 0.0)
      h = jnp.dot(y, w_ref[...], preferred_element_type=jnp.float32)
      if final:
        out_ref[...] = h + wb_ref[...]
      else:
        gs = dinv * h
        out_ref[0] = gs[:, :DSP]
        out_ref[1] = gs[:, DSP:]

  def run(p, g, dinv, b, gw, gb, ga, W, wb):
    if final:
      out_spec = pl.BlockSpec((BN, DO), lambda ph, j: (j, 0))
      out_shape = jax.ShapeDtypeStruct((N, DO), jnp.float32)
    else:
      out_spec = pl.BlockSpec((2, BN, DSP), lambda ph, j: (0, j, 0))
      out_shape = jax.ShapeDtypeStruct((2, N, DSP), jnp.float32)
    return pl.pallas_call(
        body,
        grid=(2, NB),
        in_specs=[
            pl.BlockSpec((NCORE, 2, BN, DSP), lambda ph, j: (0, 0, j, 0)),
            pl.BlockSpec((2, BN, DSP), lambda ph, j: (0, j, 0)),
            pl.BlockSpec((BN, 1), lambda ph, j: (j, 0)),
            pl.BlockSpec((1, DH), lambda ph, j: (0, 0)),
            pl.BlockSpec((1, DH), lambda ph, j: (0, 0)),
            pl.BlockSpec((1, DH), lambda ph, j: (0, 0)),
            pl.BlockSpec((1, DH), lambda ph, j: (0, 0)),
            pl.BlockSpec((DH, DO), lambda ph, j: (0, 0)),
            pl.BlockSpec((1, DO), lambda ph, j: (0, 0)),
        ],
        out_specs=out_spec,
        out_shape=out_shape,
        scratch_shapes=[
            pltpu.VMEM((1, DH), jnp.float32),
            pltpu.VMEM((1, DH), jnp.float32),
        ],
    )(p, g, dinv, b.reshape(1, DH), gw.reshape(1, DH), gb.reshape(1, DH),
      ga.reshape(1, DH), W, wb.reshape(1, DO))

  return run


_tc_mid = _make_tc_norm(DH, final=False)
_tc_fin = _make_tc_norm(DOUT, final=True)


def kernel(x, edge_index, W0, b0, gn_w0, gn_b0, gn_a0, W1, b1, gn_w1, gn_b1,
           gn_a1, W2, b2, gn_w2, gn_b2, gn_a2, fc_W, fc_b):
  src = edge_index[0]
  dst = edge_index[1]
  pad = EPAD - E
  src_p = jnp.concatenate([src, jnp.zeros((pad,), jnp.int32)]).reshape(
      NW, NCHUNKS, CHUNK)
  dst_p = jnp.concatenate([dst, jnp.full((pad,), N, jnp.int32)]).reshape(
      NW, NCHUNKS, CHUNK)

  degp = _make_deg()(dst_p)

  g0, dinv = _tc_pre(degp, x, W0)

  spmm = _make_seg_sum(DSP, DH // DSP)
  bS = jnp.stack([b0, b1])
  gwS = jnp.stack([gn_w0, gn_w1])
  gbS = jnp.stack([gn_b0, gn_b1])
  gaS = jnp.stack([gn_a0, gn_a1])
  WS = jnp.stack([W1, W2])

  def body(i, carry):
    g, _ = carry
    p = spmm(g, src_p, dst_p)
    k = jnp.minimum(i, 1)
    g_new = lax.cond(
        i < 2,
        lambda: _tc_mid(p, g, dinv, bS[k], gwS[k], gbS[k], gaS[k], WS[k],
                        bS[k]),
        lambda: g,
    )
    return (g_new, p)

  g2, p = lax.fori_loop(
      0, 3, body, (g0, jnp.zeros((NCORE, 2, N, DSP), jnp.float32)))
  return _tc_fin(p, g2, dinv, b2, gn_w2, gn_b2, gn_a2, fc_W, fc_b)


# speedup vs baseline: 1.0283x; 1.0095x over previous
import functools

import jax
import jax.numpy as jnp
from jax import lax
from jax.experimental import pallas as pl
from jax.experimental.pallas import tpu as pltpu
from jax.experimental.pallas import tpu_sc as plsc

N = 10000
E = 320000
DH = 128
DOUT = 64
EPS = 1e-5

NCORE = 2
NSUB = 16
NW = NCORE * NSUB
CHUNK = 128
NBUF = 2
EPT = 10240
EPAD = EPT * NW
NCHUNKS = EPT // CHUNK
RPT = 624
TAIL = N - RPT * NSUB
ACCR = N + 16
ZC = 16
DR = 104
DSP = 64


@functools.lru_cache(maxsize=None)
def _make_seg_sum(D, nsplit):
  mesh = plsc.VectorSubcoreMesh(core_axis_name="c", subcore_axis_name="s")

  @functools.partial(
      pl.kernel,
      out_type=jax.ShapeDtypeStruct((NCORE, nsplit, N, D), jnp.float32),
      mesh=mesh,
      scratch_types=[
          pltpu.VMEM((NCHUNKS, CHUNK), jnp.int32),
          pltpu.VMEM((NCHUNKS, CHUNK), jnp.int32),
          [pltpu.VMEM((CHUNK, D), jnp.float32)] * NBUF,
          pltpu.VMEM((ZC, D), jnp.float32),
          pltpu.VMEM((DR, D), jnp.float32),
          pltpu.VMEM_SHARED((N, D), jnp.float32),
          pltpu.VMEM_SHARED((ACCR, D), jnp.float32),
          [pltpu.SemaphoreType.DMA] * NBUF,
          [pltpu.SemaphoreType.DMA] * NBUF,
      ],
      compiler_params=pltpu.CompilerParams(use_tc_tiling_on_sc=False),
  )
  def kern(table3, src, dst, out, sidx, didx, rows, zbuf, stage, tbl, acc,
           sems, ssems):
    c = lax.axis_index("c")
    s = lax.axis_index("s")
    wid = c * NSUB + s
    last = s == NSUB - 1
    r0 = s * RPT

    pltpu.sync_copy(src.at[wid], sidx)
    pltpu.sync_copy(dst.at[wid], didx)

    def _zrow(i, carry):
      for j in range(D // 16):
        zbuf[i, pl.ds(j * 16, 16)] = jnp.zeros((16,), jnp.float32)
      return carry
    lax.fori_loop(0, ZC, _zrow, 0)

    for h in range(nsplit):
      table = table3.at[h]

      def _st(k, carry):
        rr = r0 + k * DR
        pltpu.sync_copy(table.at[pl.ds(rr, DR)], stage)
        pltpu.sync_copy(stage, tbl.at[pl.ds(rr, DR)])
        return carry
      lax.fori_loop(0, RPT // DR, _st, 0)

      @pl.when(last)
      def _():
        pltpu.sync_copy(table.at[pl.ds(RPT * NSUB, TAIL)],
                        stage.at[pl.ds(0, TAIL)])
        pltpu.sync_copy(stage.at[pl.ds(0, TAIL)],
                        tbl.at[pl.ds(RPT * NSUB, TAIL)])

      def _zcp(k, carry):
        pltpu.sync_copy(zbuf, acc.at[pl.ds(r0 + k * ZC, ZC)])
        return carry
      lax.fori_loop(0, RPT // ZC, _zcp, 0)

      @pl.when(last)
      def _():
        pltpu.sync_copy(zbuf, acc.at[pl.ds(RPT * NSUB, TAIL)])
      plsc.subcore_barrier()

      for b in range(NBUF):
        pltpu.async_copy(tbl.at[sidx.at[b]], rows[b], sems[b])

      def _pipe(it, carry):
        for b in range(NBUF):
          ch = it * NBUF + b
          pltpu.make_async_copy(tbl.at[sidx.at[0]], rows[b], sems[b]).wait()
          pltpu.async_copy(rows[b], acc.at[didx.at[ch]], ssems[b], add=True)
          bp = (b - 1) % NBUF

          @pl.when((ch >= 1) & (ch - 1 + NBUF < NCHUNKS))
          def _():
            pltpu.make_async_copy(
                rows[bp], acc.at[didx.at[0]], ssems[bp]).wait()
            pltpu.async_copy(
                tbl.at[sidx.at[ch - 1 + NBUF]], rows[bp], sems[bp])
        return carry
      lax.fori_loop(0, NCHUNKS // NBUF, _pipe, 0)
      for b in range(NBUF):
        pltpu.make_async_copy(rows[b], acc.at[didx.at[0]], ssems[b]).wait()
      plsc.subcore_barrier()

      def _dr(k, carry):
        rr = r0 + k * DR
        pltpu.sync_copy(acc.at[pl.ds(rr, DR)], stage)
        pltpu.sync_copy(stage, out.at[c, h, pl.ds(rr, DR)])
        return carry
      lax.fori_loop(0, RPT // DR, _dr, 0)

      @pl.when(last)
      def _():
        pltpu.sync_copy(acc.at[pl.ds(RPT * NSUB, TAIL)],
                        stage.at[pl.ds(0, TAIL)])
        pltpu.sync_copy(stage.at[pl.ds(0, TAIL)],
                        out.at[c, h, pl.ds(RPT * NSUB, TAIL)])

  return kern


@functools.lru_cache(maxsize=None)
def _make_deg():
  D = 16
  mesh = plsc.VectorSubcoreMesh(core_axis_name="c", subcore_axis_name="s")

  @functools.partial(
      pl.kernel,
      out_type=jax.ShapeDtypeStruct((NCORE, N, D), jnp.float32),
      mesh=mesh,
      scratch_types=[
          pltpu.VMEM((NCHUNKS, CHUNK), jnp.int32),
          pltpu.VMEM((CHUNK, D), jnp.float32),
          pltpu.VMEM((ZC, D), jnp.float32),
          pltpu.VMEM_SHARED((ACCR, D), jnp.float32),
          pltpu.SemaphoreType.DMA,
      ],
      compiler_params=pltpu.CompilerParams(use_tc_tiling_on_sc=False),
  )
  def kern(dst, out, didx, ones_r, zbuf, acc, sem):
    c = lax.axis_index("c")
    s = lax.axis_index("s")
    wid = c * NSUB + s
    last = s == NSUB - 1
    r0 = s * RPT

    pltpu.sync_copy(dst.at[wid], didx)

    def _fill(i, carry):
      zbuf[i, pl.ds(0, 16)] = jnp.zeros((16,), jnp.float32)
      return carry
    lax.fori_loop(0, ZC, _fill, 0)

    def _fill1(i, carry):
      ones_r[i, pl.ds(0, 16)] = jnp.ones((16,), jnp.float32)
      return carry
    lax.fori_loop(0, CHUNK, _fill1, 0)

    def _zcp(k, carry):
      pltpu.sync_copy(zbuf, acc.at[pl.ds(r0 + k * ZC, ZC)])
      return carry
    lax.fori_loop(0, RPT // ZC, _zcp, 0)

    @pl.when(last)
    def _():
      pltpu.sync_copy(zbuf, acc.at[pl.ds(RPT * NSUB, TAIL)])
    plsc.subcore_barrier()

    FIRE = 16

    def _batch(g, carry):
      for b in range(FIRE):
        pltpu.async_copy(ones_r, acc.at[didx.at[g * FIRE + b]], sem,
                         add=True)
      for b in range(FIRE):
        pltpu.make_async_copy(ones_r, acc.at[didx.at[0]], sem).wait()
      return carry
    lax.fori_loop(0, NCHUNKS // FIRE, _batch, 0)
    plsc.subcore_barrier()

    def _dr(k, carry):
      rr = r0 + k * ZC
      pltpu.sync_copy(acc.at[pl.ds(rr, ZC)], zbuf)
      pltpu.sync_copy(zbuf, out.at[c, pl.ds(rr, ZC)])
      return carry
    lax.fori_loop(0, RPT // ZC, _dr, 0)

    @pl.when(last)
    def _():
      pltpu.sync_copy(acc.at[pl.ds(RPT * NSUB, TAIL)], zbuf)
      pltpu.sync_copy(zbuf, out.at[c, pl.ds(RPT * NSUB, TAIL)])

  return kern


BN = 2000
NB = N // BN


def _tc_pre(degp, x, W0):
  def body(deg_ref, x_ref, w_ref, g_ref, dinv_ref):
    deg = deg_ref[0, :, 0:1] + deg_ref[1, :, 0:1] + 1.0
    dinv = lax.rsqrt(deg)
    h = jnp.dot(x_ref[...], w_ref[...], preferred_element_type=jnp.float32)
    gs = dinv * h
    g_ref[0] = gs[:, :DSP]
    g_ref[1] = gs[:, DSP:]
    dinv_ref[...] = dinv

  return pl.pallas_call(
      body,
      grid=(NB,),
      in_specs=[
          pl.BlockSpec((NCORE, BN, 16), lambda j: (0, j, 0)),
          pl.BlockSpec((BN, DH), lambda j: (j, 0)),
          pl.BlockSpec((DH, DH), lambda j: (0, 0)),
      ],
      out_specs=[
          pl.BlockSpec((2, BN, DSP), lambda j: (0, j, 0)),
          pl.BlockSpec((BN, 1), lambda j: (j, 0)),
      ],
      out_shape=[
          jax.ShapeDtypeStruct((2, N, DSP), jnp.float32),
          jax.ShapeDtypeStruct((N, 1), jnp.float32),
      ],
  )(degp, x, W0)


def _make_tc_norm(DO, final):
  def body(p_ref, g_ref, dinv_ref, b_ref, gw_ref, gb_ref, ga_ref, w_ref,
           wb_ref, out_ref, S1, S2):
    ph = pl.program_id(0)
    j = pl.program_id(1)
    dinv = dinv_ref[...]
    psum = p_ref[0] + p_ref[1]
    pcat = jnp.concatenate([psum[0] + g_ref[0], psum[1] + g_ref[1]], axis=1)
    t = dinv * pcat + b_ref[...]

    @pl.when(ph == 0)
    def _():
      s1 = jnp.sum(t, axis=0, keepdims=True)
      s2 = jnp.sum(t * t, axis=0, keepdims=True)

      @pl.when(j == 0)
      def _():
        S1[...] = s1
        S2[...] = s2

      @pl.when(j != 0)
      def _():
        S1[...] += s1
        S2[...] += s2

    @pl.when(ph == 1)
    def _():
      m = S1[...] * (1.0 / N)
      ex2 = S2[...] * (1.0 / N)
      am = ga_ref[...] * m
      var = ex2 - 2.0 * m * am + am * am
      y = gw_ref[...] * (t - am) * lax.rsqrt(var + EPS) + gb_ref[...]
      y = jnp.maximum(y, 0.0)
      h = jnp.dot(y, w_ref[...], preferred_element_type=jnp.float32)
      if final:
        out_ref[...] = h + wb_ref[...]
      else:
        gs = dinv * h
        out_ref[0] = gs[:, :DSP]
        out_ref[1] = gs[:, DSP:]

  def run(p, g, dinv, b, gw, gb, ga, W, wb):
    if final:
      out_spec = pl.BlockSpec((BN, DO), lambda ph, j: (j * ph, 0))
      out_shape = jax.ShapeDtypeStruct((N, DO), jnp.float32)
    else:
      out_spec = pl.BlockSpec((2, BN, DSP), lambda ph, j: (0, j * ph, 0))
      out_shape = jax.ShapeDtypeStruct((2, N, DSP), jnp.float32)
    return pl.pallas_call(
        body,
        grid=(2, NB),
        in_specs=[
            pl.BlockSpec((NCORE, 2, BN, DSP), lambda ph, j: (0, 0, j, 0)),
            pl.BlockSpec((2, BN, DSP), lambda ph, j: (0, j, 0)),
            pl.BlockSpec((BN, 1), lambda ph, j: (j, 0)),
            pl.BlockSpec((1, DH), lambda ph, j: (0, 0)),
            pl.BlockSpec((1, DH), lambda ph, j: (0, 0)),
            pl.BlockSpec((1, DH), lambda ph, j: (0, 0)),
            pl.BlockSpec((1, DH), lambda ph, j: (0, 0)),
            pl.BlockSpec((DH, DO), lambda ph, j: (0, 0)),
            pl.BlockSpec((1, DO), lambda ph, j: (0, 0)),
        ],
        out_specs=out_spec,
        out_shape=out_shape,
        scratch_shapes=[
            pltpu.VMEM((1, DH), jnp.float32),
            pltpu.VMEM((1, DH), jnp.float32),
        ],
    )(p, g, dinv, b.reshape(1, DH), gw.reshape(1, DH), gb.reshape(1, DH),
      ga.reshape(1, DH), W, wb.reshape(1, DO))

  return run


_tc_mid = _make_tc_norm(DH, final=False)
_tc_fin = _make_tc_norm(DOUT, final=True)


def kernel(x, edge_index, W0, b0, gn_w0, gn_b0, gn_a0, W1, b1, gn_w1, gn_b1,
           gn_a1, W2, b2, gn_w2, gn_b2, gn_a2, fc_W, fc_b):
  src = edge_index[0]
  dst = edge_index[1]
  pad = EPAD - E
  src_p = jnp.concatenate([src, jnp.zeros((pad,), jnp.int32)]).reshape(
      NW, NCHUNKS, CHUNK)
  dst_p = jnp.concatenate([dst, jnp.full((pad,), N, jnp.int32)]).reshape(
      NW, NCHUNKS, CHUNK)

  degp = _make_deg()(dst_p)

  g0, dinv = _tc_pre(degp, x, W0)

  spmm = _make_seg_sum(DSP, DH // DSP)
  bS = jnp.stack([b0, b1])
  gwS = jnp.stack([gn_w0, gn_w1])
  gbS = jnp.stack([gn_b0, gn_b1])
  gaS = jnp.stack([gn_a0, gn_a1])
  WS = jnp.stack([W1, W2])

  def body(i, carry):
    g, _ = carry
    p = spmm(g, src_p, dst_p)
    k = jnp.minimum(i, 1)
    g_new = lax.cond(
        i < 2,
        lambda: _tc_mid(p, g, dinv, bS[k], gwS[k], gbS[k], gaS[k], WS[k],
                        bS[k]),
        lambda: g,
    )
    return (g_new, p)

  g2, p = lax.fori_loop(
      0, 3, body, (g0, jnp.zeros((NCORE, 2, N, DSP), jnp.float32)))
  return _tc_fin(p, g2, dinv, b2, gn_w2, gn_b2, gn_a2, fc_W, fc_b)


# speedup vs baseline: 1.0291x; 1.0008x over previous
import functools

import jax
import jax.numpy as jnp
from jax import lax
from jax.experimental import pallas as pl
from jax.experimental.pallas import tpu as pltpu
from jax.experimental.pallas import tpu_sc as plsc

N = 10000
E = 320000
DH = 128
DOUT = 64
EPS = 1e-5

NCORE = 2
NSUB = 16
NW = NCORE * NSUB
CHUNK = 128
NBUF = 2
EPT = 10240
EPAD = EPT * NW
NCHUNKS = EPT // CHUNK
RPT = 624
TAIL = N - RPT * NSUB
ACCR = N + 16
ZC = 16
DR = 104
DSP = 64


@functools.lru_cache(maxsize=None)
def _make_seg_sum(D, nsplit):
  mesh = plsc.VectorSubcoreMesh(core_axis_name="c", subcore_axis_name="s")

  @functools.partial(
      pl.kernel,
      out_type=jax.ShapeDtypeStruct((NCORE, nsplit, N, D), jnp.float32),
      mesh=mesh,
      scratch_types=[
          pltpu.VMEM((NCHUNKS, CHUNK), jnp.int32),
          pltpu.VMEM((NCHUNKS, CHUNK), jnp.int32),
          [pltpu.VMEM((CHUNK, D), jnp.float32)] * NBUF,
          pltpu.VMEM((ZC, D), jnp.float32),
          pltpu.VMEM((DR, D), jnp.float32),
          pltpu.VMEM_SHARED((N, D), jnp.float32),
          pltpu.VMEM_SHARED((ACCR, D), jnp.float32),
          [pltpu.SemaphoreType.DMA] * NBUF,
          [pltpu.SemaphoreType.DMA] * NBUF,
      ],
      compiler_params=pltpu.CompilerParams(use_tc_tiling_on_sc=False),
  )
  def kern(table3, src, dst, out, sidx, didx, rows, zbuf, stage, tbl, acc,
           sems, ssems):
    c = lax.axis_index("c")
    s = lax.axis_index("s")
    wid = c * NSUB + s
    last = s == NSUB - 1
    r0 = s * RPT

    pltpu.sync_copy(src.at[wid], sidx)
    pltpu.sync_copy(dst.at[wid], didx)

    def _zrow(i, carry):
      for j in range(D // 16):
        zbuf[i, pl.ds(j * 16, 16)] = jnp.zeros((16,), jnp.float32)
      return carry
    lax.fori_loop(0, ZC, _zrow, 0)

    for h in range(nsplit):
      table = table3.at[h]

      def _st(k, carry):
        rr = r0 + k * DR
        pltpu.sync_copy(table.at[pl.ds(rr, DR)], stage)
        pltpu.sync_copy(stage, tbl.at[pl.ds(rr, DR)])
        return carry
      lax.fori_loop(0, RPT // DR, _st, 0)

      @pl.when(last)
      def _():
        pltpu.sync_copy(table.at[pl.ds(RPT * NSUB, TAIL)],
                        stage.at[pl.ds(0, TAIL)])
        pltpu.sync_copy(stage.at[pl.ds(0, TAIL)],
                        tbl.at[pl.ds(RPT * NSUB, TAIL)])

      def _zcp(k, carry):
        pltpu.sync_copy(zbuf, acc.at[pl.ds(r0 + k * ZC, ZC)])
        return carry
      lax.fori_loop(0, RPT // ZC, _zcp, 0)

      @pl.when(last)
      def _():
        pltpu.sync_copy(zbuf, acc.at[pl.ds(RPT * NSUB, TAIL)])
      plsc.subcore_barrier()

      for b in range(NBUF):
        pltpu.async_copy(tbl.at[sidx.at[b]], rows[b], sems[b])

      UNROLL = 2 * NBUF

      def _pipe(it, carry):
        for bb in range(UNROLL):
          b = bb % NBUF
          ch = it * UNROLL + bb
          pltpu.make_async_copy(tbl.at[sidx.at[0]], rows[b], sems[b]).wait()
          pltpu.async_copy(rows[b], acc.at[didx.at[ch]], ssems[b], add=True)
          bp = (b - 1) % NBUF

          @pl.when((ch >= 1) & (ch - 1 + NBUF < NCHUNKS))
          def _():
            pltpu.make_async_copy(
                rows[bp], acc.at[didx.at[0]], ssems[bp]).wait()
            pltpu.async_copy(
                tbl.at[sidx.at[ch - 1 + NBUF]], rows[bp], sems[bp])
        return carry
      lax.fori_loop(0, NCHUNKS // UNROLL, _pipe, 0)
      for b in range(NBUF):
        pltpu.make_async_copy(rows[b], acc.at[didx.at[0]], ssems[b]).wait()
      plsc.subcore_barrier()

      def _dr(k, carry):
        rr = r0 + k * DR
        pltpu.sync_copy(acc.at[pl.ds(rr, DR)], stage)
        pltpu.sync_copy(stage, out.at[c, h, pl.ds(rr, DR)])
        return carry
      lax.fori_loop(0, RPT // DR, _dr, 0)

      @pl.when(last)
      def _():
        pltpu.sync_copy(acc.at[pl.ds(RPT * NSUB, TAIL)],
                        stage.at[pl.ds(0, TAIL)])
        pltpu.sync_copy(stage.at[pl.ds(0, TAIL)],
                        out.at[c, h, pl.ds(RPT * NSUB, TAIL)])

  return kern


@functools.lru_cache(maxsize=None)
def _make_deg():
  D = 16
  mesh = plsc.VectorSubcoreMesh(core_axis_name="c", subcore_axis_name="s")

  @functools.partial(
      pl.kernel,
      out_type=jax.ShapeDtypeStruct((NCORE, N, D), jnp.float32),
      mesh=mesh,
      scratch_types=[
          pltpu.VMEM((NCHUNKS, CHUNK), jnp.int32),
          pltpu.VMEM((CHUNK, D), jnp.float32),
          pltpu.VMEM((ZC, D), jnp.float32),
          pltpu.VMEM_SHARED((ACCR, D), jnp.float32),
          pltpu.SemaphoreType.DMA,
      ],
      compiler_params=pltpu.CompilerParams(use_tc_tiling_on_sc=False),
  )
  def kern(dst, out, didx, ones_r, zbuf, acc, sem):
    c = lax.axis_index("c")
    s = lax.axis_index("s")
    wid = c * NSUB + s
    last = s == NSUB - 1
    r0 = s * RPT

    pltpu.sync_copy(dst.at[wid], didx)

    def _fill(i, carry):
      zbuf[i, pl.ds(0, 16)] = jnp.zeros((16,), jnp.float32)
      return carry
    lax.fori_loop(0, ZC, _fill, 0)

    def _fill1(i, carry):
      ones_r[i, pl.ds(0, 16)] = jnp.ones((16,), jnp.float32)
      return carry
    lax.fori_loop(0, CHUNK, _fill1, 0)

    def _zcp(k, carry):
      pltpu.sync_copy(zbuf, acc.at[pl.ds(r0 + k * ZC, ZC)])
      return carry
    lax.fori_loop(0, RPT // ZC, _zcp, 0)

    @pl.when(last)
    def _():
      pltpu.sync_copy(zbuf, acc.at[pl.ds(RPT * NSUB, TAIL)])
    plsc.subcore_barrier()

    FIRE = 16

    def _batch(g, carry):
      for b in range(FIRE):
        pltpu.async_copy(ones_r, acc.at[didx.at[g * FIRE + b]], sem,
                         add=True)
      for b in range(FIRE):
        pltpu.make_async_copy(ones_r, acc.at[didx.at[0]], sem).wait()
      return carry
    lax.fori_loop(0, NCHUNKS // FIRE, _batch, 0)
    plsc.subcore_barrier()

    def _dr(k, carry):
      rr = r0 + k * ZC
      pltpu.sync_copy(acc.at[pl.ds(rr, ZC)], zbuf)
      pltpu.sync_copy(zbuf, out.at[c, pl.ds(rr, ZC)])
      return carry
    lax.fori_loop(0, RPT // ZC, _dr, 0)

    @pl.when(last)
    def _():
      pltpu.sync_copy(acc.at[pl.ds(RPT * NSUB, TAIL)], zbuf)
      pltpu.sync_copy(zbuf, out.at[c, pl.ds(RPT * NSUB, TAIL)])

  return kern


BN = 2000
NB = N // BN


def _tc_pre(degp, x, W0):
  def body(deg_ref, x_ref, w_ref, g_ref, dinv_ref):
    deg = deg_ref[0, :, 0:1] + deg_ref[1, :, 0:1] + 1.0
    dinv = lax.rsqrt(deg)
    h = jnp.dot(x_ref[...], w_ref[...], preferred_element_type=jnp.float32)
    gs = dinv * h
    g_ref[0] = gs[:, :DSP]
    g_ref[1] = gs[:, DSP:]
    dinv_ref[...] = dinv

  return pl.pallas_call(
      body,
      grid=(NB,),
      in_specs=[
          pl.BlockSpec((NCORE, BN, 16), lambda j: (0, j, 0)),
          pl.BlockSpec((BN, DH), lambda j: (j, 0)),
          pl.BlockSpec((DH, DH), lambda j: (0, 0)),
      ],
      out_specs=[
          pl.BlockSpec((2, BN, DSP), lambda j: (0, j, 0)),
          pl.BlockSpec((BN, 1), lambda j: (j, 0)),
      ],
      out_shape=[
          jax.ShapeDtypeStruct((2, N, DSP), jnp.float32),
          jax.ShapeDtypeStruct((N, 1), jnp.float32),
      ],
  )(degp, x, W0)


def _make_tc_norm(DO, final):
  def body(p_ref, g_ref, dinv_ref, b_ref, gw_ref, gb_ref, ga_ref, w_ref,
           wb_ref, out_ref, S1, S2):
    ph = pl.program_id(0)
    j = pl.program_id(1)
    dinv = dinv_ref[...]
    psum = p_ref[0] + p_ref[1]
    pcat = jnp.concatenate([psum[0] + g_ref[0], psum[1] + g_ref[1]], axis=1)
    t = dinv * pcat + b_ref[...]

    @pl.when(ph == 0)
    def _():
      s1 = jnp.sum(t, axis=0, keepdims=True)
      s2 = jnp.sum(t * t, axis=0, keepdims=True)

      @pl.when(j == 0)
      def _():
        S1[...] = s1
        S2[...] = s2

      @pl.when(j != 0)
      def _():
        S1[...] += s1
        S2[...] += s2

    @pl.when(ph == 1)
    def _():
      m = S1[...] * (1.0 / N)
      ex2 = S2[...] * (1.0 / N)
      am = ga_ref[...] * m
      var = ex2 - 2.0 * m * am + am * am
      y = gw_ref[...] * (t - am) * lax.rsqrt(var + EPS) + gb_ref[...]
      y = jnp.maximum(y, 0.0)
      h = jnp.dot(y, w_ref[...], preferred_element_type=jnp.float32)
      if final:
        out_ref[...] = h + wb_ref[...]
      else:
        gs = dinv * h
        out_ref[0] = gs[:, :DSP]
        out_ref[1] = gs[:, DSP:]

  def run(p, g, dinv, b, gw, gb, ga, W, wb):
    if final:
      out_spec = pl.BlockSpec((BN, DO), lambda ph, j: (j * ph, 0))
      out_shape = jax.ShapeDtypeStruct((N, DO), jnp.float32)
    else:
      out_spec = pl.BlockSpec((2, BN, DSP), lambda ph, j: (0, j * ph, 0))
      out_shape = jax.ShapeDtypeStruct((2, N, DSP), jnp.float32)
    return pl.pallas_call(
        body,
        grid=(2, NB),
        in_specs=[
            pl.BlockSpec((NCORE, 2, BN, DSP), lambda ph, j: (0, 0, j, 0)),
            pl.BlockSpec((2, BN, DSP), lambda ph, j: (0, j, 0)),
            pl.BlockSpec((BN, 1), lambda ph, j: (j, 0)),
            pl.BlockSpec((1, DH), lambda ph, j: (0, 0)),
            pl.BlockSpec((1, DH), lambda ph, j: (0, 0)),
            pl.BlockSpec((1, DH), lambda ph, j: (0, 0)),
            pl.BlockSpec((1, DH), lambda ph, j: (0, 0)),
            pl.BlockSpec((DH, DO), lambda ph, j: (0, 0)),
            pl.BlockSpec((1, DO), lambda ph, j: (0, 0)),
        ],
        out_specs=out_spec,
        out_shape=out_shape,
        scratch_shapes=[
            pltpu.VMEM((1, DH), jnp.float32),
            pltpu.VMEM((1, DH), jnp.float32),
        ],
    )(p, g, dinv, b.reshape(1, DH), gw.reshape(1, DH), gb.reshape(1, DH),
      ga.reshape(1, DH), W, wb.reshape(1, DO))

  return run


_tc_mid = _make_tc_norm(DH, final=False)
_tc_fin = _make_tc_norm(DOUT, final=True)


def kernel(x, edge_index, W0, b0, gn_w0, gn_b0, gn_a0, W1, b1, gn_w1, gn_b1,
           gn_a1, W2, b2, gn_w2, gn_b2, gn_a2, fc_W, fc_b):
  src = edge_index[0]
  dst = edge_index[1]
  pad = EPAD - E
  src_p = jnp.concatenate([src, jnp.zeros((pad,), jnp.int32)]).reshape(
      NW, NCHUNKS, CHUNK)
  dst_p = jnp.concatenate([dst, jnp.full((pad,), N, jnp.int32)]).reshape(
      NW, NCHUNKS, CHUNK)

  degp = _make_deg()(dst_p)

  g0, dinv = _tc_pre(degp, x, W0)

  spmm = _make_seg_sum(DSP, DH // DSP)
  bS = jnp.stack([b0, b1])
  gwS = jnp.stack([gn_w0, gn_w1])
  gbS = jnp.stack([gn_b0, gn_b1])
  gaS = jnp.stack([gn_a0, gn_a1])
  WS = jnp.stack([W1, W2])

  def body(i, carry):
    g, _ = carry
    p = spmm(g, src_p, dst_p)
    k = jnp.minimum(i, 1)
    g_new = lax.cond(
        i < 2,
        lambda: _tc_mid(p, g, dinv, bS[k], gwS[k], gbS[k], gaS[k], WS[k],
                        bS[k]),
        lambda: g,
    )
    return (g_new, p)

  g2, p = lax.fori_loop(
      0, 3, body, (g0, jnp.zeros((NCORE, 2, N, DSP), jnp.float32)))
  return _tc_fin(p, g2, dinv, b2, gn_w2, gn_b2, gn_a2, fc_W, fc_b)
